# baseline (speedup 1.0000x reference)
import jax
import jax.numpy as jnp
from jax import lax
from jax.experimental import pallas as pl
from jax.experimental.pallas import tpu as pltpu
from jax.experimental.pallas import tpu_sc as plsc

B = 4096
N_DISC, N_CONT = 6, 4
N_FEAT = N_DISC + N_CONT
V1 = 100001
D = 32
NC, NS = 2, 16
NW = NC * NS
BW = B // NW
NK = BW // 16
NG = 3
BK = 512


def _sc_body(*refs):
    idx = refs[0]
    tab = refs[1]
    out = refs[2]
    istage = refs[3]
    ring = refs[4]
    gslab = refs[5]
    gsems = refs[6]
    wsem = refs[7]

    wid = lax.axis_index("s") * NC + lax.axis_index("c")
    base = pl.multiple_of(wid * BW, BW)

    def fire_chunk(i, k):
        cvec = istage[0, pl.ds(pl.multiple_of(k * 16, 16), 16)]
        half = lax.rem(k, 2)
        for j in range(16):
            s = cvec[j] + 1
            blk = pl.multiple_of((s // 8) * 8, 8)
            slot = pl.multiple_of(half * BW + j * 8, 8)
            pltpu.async_copy(tab.at[i, pl.ds(blk, 8)],
                             ring.at[pl.ds(slot, 8)], gsems.at[half])

    def retire_chunk(il, i, k):
        cvec = istage[0, pl.ds(pl.multiple_of(k * 16, 16), 16)]
        half = lax.rem(k, 2)
        pltpu.make_async_copy(tab.at[i, pl.ds(0, BW)],
                              ring.at[pl.ds(pl.multiple_of(half * BW, 8),
                                            BW)],
                              gsems.at[half]).wait()
        base_r = half * BW
        for j in range(16):
            s = cvec[j] + 1
            rem = lax.rem(s, 8)
            row = base_r + j * 8 + rem
            gslab[il, k * 16 + j, pl.ds(0, 16)] = ring[row, pl.ds(0, 16)]
            gslab[il, k * 16 + j, pl.ds(16, 16)] = ring[row, pl.ds(16, 16)]

    prev_dst = None
    for h in range(2):
        if prev_dst is not None:
            pltpu.make_async_copy(gslab, prev_dst, wsem).wait()

        def iloop(il, carry, h=h):
            i = h * (N_DISC // 2) + il
            pltpu.sync_copy(idx.at[i, wid], istage)
            fire_chunk(i, 0)

            def kbody(k, c):
                fire_chunk(i, k)
                retire_chunk(il, i, k - 1)
                return c

            lax.fori_loop(1, NK, kbody, None)
            retire_chunk(il, i, NK - 1)
            return carry

        lax.fori_loop(0, N_DISC // 2, iloop, None)
        dst = out.at[pl.ds(h * (N_DISC // 2), N_DISC // 2),
                     pl.ds(base, BW)]
        pltpu.async_copy(gslab, dst, wsem)
        prev_dst = dst

    pltpu.make_async_copy(gslab, prev_dst, wsem).wait()


CB = 256
NFULL = V1 // CB
NT = -(-NFULL // NW)
VTAIL = V1 - NFULL * CB
VTAIL8 = -(-VTAIL // 8) * 8
TW = NFULL - (NT - 1) * NW


def _tr_body(tabt, out, tbuf, obuf, isems, osems):
    wid = lax.axis_index("s") * NC + lax.axis_index("c")
    iot = lax.iota(jnp.int32, 16)

    def transpose_block(p, ncc):
        def cc_body(cc, carry):
            pv = jnp.full((16,), 0, jnp.int32) + p
            cv = jnp.full((16,), 0, jnp.int32) + cc
            g1 = plsc.load_gather(tbuf, [pv, iot, cv])
            g2 = plsc.load_gather(tbuf, [pv, iot + 16, cv])
            obuf[p, cc, pl.ds(0, 16)] = g1
            obuf[p, cc, pl.ds(16, 16)] = g2
            return carry

        lax.fori_loop(0, ncc, cc_body, None)

    for i in range(N_DISC):
        def fire_in(t, i=i):
            p = lax.rem(t, 2)
            c0 = pl.multiple_of((t * NW + wid) * CB, CB)
            pltpu.async_copy(tabt.at[i, :, pl.ds(c0, CB)], tbuf.at[p],
                             isems.at[p])

        fire_in(0)

        def tbody(t, carry, i=i):
            p = lax.rem(t, 2)
            c0 = pl.multiple_of((t * NW + wid) * CB, CB)

            @pl.when(((t + 1) * NW + wid) < NFULL)
            def _():
                fire_in(t + 1)

            @pl.when((t * NW + wid) < NFULL)
            def _():
                pltpu.make_async_copy(tabt.at[i, :, pl.ds(0, CB)],
                                      tbuf.at[p], isems.at[p]).wait()

                @pl.when(t >= 2)
                def _():
                    pltpu.make_async_copy(obuf.at[p],
                                          out.at[i, pl.ds(0, CB)],
                                          osems.at[p]).wait()

                transpose_block(p, CB)
                pltpu.async_copy(obuf.at[p], out.at[i, pl.ds(c0, CB)],
                                 osems.at[p])
            return carry

        lax.fori_loop(0, NT, tbody, None)

        for t in (NT - 2, NT - 1):
            @pl.when((t * NW + wid) < NFULL)
            def _(t=t, i=i):
                pltpu.make_async_copy(obuf.at[t % 2],
                                      out.at[i, pl.ds(0, CB)],
                                      osems.at[t % 2]).wait()

        @pl.when(wid == TW)
        def _tail(i=i):
            c0 = pl.multiple_of(NFULL * CB, CB)
            pltpu.sync_copy(tabt.at[i, :, pl.ds(c0, CB)], tbuf.at[0])
            transpose_block(jnp.int32(0), VTAIL8)
            pltpu.sync_copy(obuf.at[0, pl.ds(0, VTAIL8)],
                            out.at[i, pl.ds(c0, VTAIL8)])


def _tc_body(c0, c1, c2, w0, b0, w1, b1, w2, b2, o0, o1, o2):
    for c, w, bb, o in ((c0, w0, b0, o0), (c1, w1, b1, o1),
                        (c2, w2, b2, o2)):
        cv = c[:, 0, :]
        for j in range(N_CONT):
            o[:, j, :] = (cv[:, j][:, None] * w[j][None, :]
                          + bb[j][None, :])


@jax.jit
def _impl(qoe_d, ch_d, fu_d, qoe_c, ch_c, fu_c,
          qoe_tab, ch_tab, fu_tab,
          qoe_w, qoe_b, ch_w, ch_b, fu_w, fu_b):
    idxT = [d.reshape(B, N_DISC).T.reshape(N_DISC, NW, 1, BW)
            for d in (qoe_d, ch_d, fu_d)]

    mesh = plsc.VectorSubcoreMesh(core_axis_name="c", subcore_axis_name="s")
    out_t = jax.ShapeDtypeStruct((N_DISC, B, D), jnp.float32)
    scratch = (
        [pltpu.VMEM((1, BW), jnp.int32)]
        + [pltpu.VMEM((2 * BW, D), jnp.float32)]
        + [pltpu.VMEM((N_DISC // 2, BW, D), jnp.float32)]
        + [pltpu.SemaphoreType.DMA((2,)), pltpu.SemaphoreType.DMA]
    )
    run = pl.kernel(_sc_body, mesh=mesh, out_type=out_t,
                    scratch_types=scratch,
                    compiler_params=pltpu.CompilerParams(
                        use_tc_tiling_on_sc=True,
                        needs_layout_passes=False))
    retile = pl.kernel(
        _tr_body, mesh=mesh,
        out_type=jax.ShapeDtypeStruct((N_DISC, V1, D), jnp.float32),
        scratch_types=[
            pltpu.VMEM((2, D, CB), jnp.float32),
            pltpu.VMEM((2, CB, D), jnp.float32),
            pltpu.SemaphoreType.DMA((2,)),
            pltpu.SemaphoreType.DMA((2,)),
        ],
        compiler_params=pltpu.CompilerParams(
            use_tc_tiling_on_sc=True,
            needs_layout_passes=False))
    discs = [run(idxT[g], retile(t.transpose(0, 2, 1)))
             for g, t in enumerate((qoe_tab, ch_tab, fu_tab))]

    grid = (B // BK,)
    cspec = pl.BlockSpec((BK, 1, N_CONT), lambda b: (b, 0, 0))
    wspec = pl.BlockSpec((N_CONT, D), lambda b: (0, 0))
    ospec = pl.BlockSpec((BK, N_CONT, D), lambda b: (b, 0, 0))
    conts = pl.pallas_call(
        _tc_body,
        grid=grid,
        in_specs=[cspec] * 3 + [wspec] * 6,
        out_specs=[ospec] * 3,
        out_shape=[jax.ShapeDtypeStruct((B, N_CONT, D), jnp.float32)] * 3,
    )(qoe_c, ch_c, fu_c, qoe_w, qoe_b, ch_w, ch_b, fu_w, fu_b)
    return tuple(
        jnp.concatenate([d.transpose(1, 0, 2), c], axis=1)
        .reshape(B, 1, N_FEAT, D)
        for d, c in zip(discs, conts))


def kernel(batch_feature_tensor_target_QOE_discrete,
           batch_feature_tensor_target_CHONGHE_discrete,
           batch_feature_tensor_target_FUFEI_discrete,
           batch_feature_tensor_target_QOE_continue,
           batch_feature_tensor_target_CHONGHE_continue,
           batch_feature_tensor_target_FUFEI_continue,
           qoe_tables, chonghe_tables, fufei_tables,
           qoe_cont_w, qoe_cont_b, chonghe_cont_w, chonghe_cont_b,
           fufei_cont_w, fufei_cont_b):
    return _impl(batch_feature_tensor_target_QOE_discrete,
                 batch_feature_tensor_target_CHONGHE_discrete,
                 batch_feature_tensor_target_FUFEI_discrete,
                 batch_feature_tensor_target_QOE_continue,
                 batch_feature_tensor_target_CHONGHE_continue,
                 batch_feature_tensor_target_FUFEI_continue,
                 qoe_tables, chonghe_tables, fufei_tables,
                 qoe_cont_w, qoe_cont_b, chonghe_cont_w, chonghe_cont_b,
                 fufei_cont_w, fufei_cont_b)


# speedup vs baseline: 2.2095x; 2.2095x over previous
import jax
import jax.numpy as jnp
from jax import lax
from jax.experimental import pallas as pl
from jax.experimental.pallas import tpu as pltpu
from jax.experimental.pallas import tpu_sc as plsc

B = 4096
N_DISC, N_CONT = 6, 4
N_FEAT = N_DISC + N_CONT
V1 = 100001
D = 32
NC, NS = 2, 16
NW = NC * NS
BW = B // NW
NK = BW // 16
NG = 3
BK = 512


def _sc_body(*refs):
    idx = refs[0]
    tab = refs[1]
    out = refs[2]
    istage = refs[3]
    ring = refs[4]
    gslab = refs[5]
    gsems = refs[6]
    wsem = refs[7]

    wid = lax.axis_index("s") * NC + lax.axis_index("c")
    base = pl.multiple_of(wid * BW, BW)

    def fire_chunk(i, k):
        cvec = istage[0, pl.ds(pl.multiple_of(k * 16, 16), 16)]
        half = lax.rem(k, 2)
        for j in range(16):
            s = cvec[j] + 1
            blk = pl.multiple_of((s // 8) * 8, 8)
            slot = pl.multiple_of(half * BW + j * 8, 8)
            pltpu.async_copy(tab.at[i, pl.ds(blk, 8)],
                             ring.at[pl.ds(slot, 8)], gsems.at[half])

    def retire_chunk(il, i, k):
        cvec = istage[0, pl.ds(pl.multiple_of(k * 16, 16), 16)]
        half = lax.rem(k, 2)
        pltpu.make_async_copy(tab.at[i, pl.ds(0, BW)],
                              ring.at[pl.ds(pl.multiple_of(half * BW, 8),
                                            BW)],
                              gsems.at[half]).wait()
        base_r = half * BW
        for j in range(16):
            s = cvec[j] + 1
            rem = lax.rem(s, 8)
            row = base_r + j * 8 + rem
            gslab[il, k * 16 + j, pl.ds(0, 16)] = ring[row, pl.ds(0, 16)]
            gslab[il, k * 16 + j, pl.ds(16, 16)] = ring[row, pl.ds(16, 16)]

    prev_dst = None
    for h in range(2):
        if prev_dst is not None:
            pltpu.make_async_copy(gslab, prev_dst, wsem).wait()

        def iloop(il, carry, h=h):
            i = h * (N_DISC // 2) + il
            pltpu.sync_copy(idx.at[i, wid], istage)
            fire_chunk(i, 0)

            def kbody(k, c):
                fire_chunk(i, k)
                retire_chunk(il, i, k - 1)
                return c

            lax.fori_loop(1, NK, kbody, None)
            retire_chunk(il, i, NK - 1)
            return carry

        lax.fori_loop(0, N_DISC // 2, iloop, None)
        dst = out.at[pl.ds(h * (N_DISC // 2), N_DISC // 2),
                     pl.ds(base, BW)]
        pltpu.async_copy(gslab, dst, wsem)
        prev_dst = dst

    pltpu.make_async_copy(gslab, prev_dst, wsem).wait()


def _tc_body(c0, c1, c2, w0, b0, w1, b1, w2, b2, o0, o1, o2):
    for c, w, bb, o in ((c0, w0, b0, o0), (c1, w1, b1, o1),
                        (c2, w2, b2, o2)):
        cv = c[:, 0, :]
        for j in range(N_CONT):
            o[:, j, :] = (cv[:, j][:, None] * w[j][None, :]
                          + bb[j][None, :])


@jax.jit
def _impl(qoe_d, ch_d, fu_d, qoe_c, ch_c, fu_c,
          qoe_tab, ch_tab, fu_tab,
          qoe_w, qoe_b, ch_w, ch_b, fu_w, fu_b):
    idxT = [d.reshape(B, N_DISC).T.reshape(N_DISC, NW, 1, BW)
            for d in (qoe_d, ch_d, fu_d)]

    mesh = plsc.VectorSubcoreMesh(core_axis_name="c", subcore_axis_name="s")
    out_t = jax.ShapeDtypeStruct((N_DISC, B, D), jnp.float32)
    scratch = (
        [pltpu.VMEM((1, BW), jnp.int32)]
        + [pltpu.VMEM((2 * BW, D), jnp.float32)]
        + [pltpu.VMEM((N_DISC // 2, BW, D), jnp.float32)]
        + [pltpu.SemaphoreType.DMA((2,)), pltpu.SemaphoreType.DMA]
    )
    run = pl.kernel(_sc_body, mesh=mesh, out_type=out_t,
                    scratch_types=scratch,
                    compiler_params=pltpu.CompilerParams(
                        use_tc_tiling_on_sc=True,
                        needs_layout_passes=False))
    discs = [run(idxT[g], t)
             for g, t in enumerate((qoe_tab, ch_tab, fu_tab))]

    grid = (B // BK,)
    cspec = pl.BlockSpec((BK, 1, N_CONT), lambda b: (b, 0, 0))
    wspec = pl.BlockSpec((N_CONT, D), lambda b: (0, 0))
    ospec = pl.BlockSpec((BK, N_CONT, D), lambda b: (b, 0, 0))
    conts = pl.pallas_call(
        _tc_body,
        grid=grid,
        in_specs=[cspec] * 3 + [wspec] * 6,
        out_specs=[ospec] * 3,
        out_shape=[jax.ShapeDtypeStruct((B, N_CONT, D), jnp.float32)] * 3,
    )(qoe_c, ch_c, fu_c, qoe_w, qoe_b, ch_w, ch_b, fu_w, fu_b)
    return tuple(
        jnp.concatenate([d.transpose(1, 0, 2), c], axis=1)
        .reshape(B, 1, N_FEAT, D)
        for d, c in zip(discs, conts))


def kernel(batch_feature_tensor_target_QOE_discrete,
           batch_feature_tensor_target_CHONGHE_discrete,
           batch_feature_tensor_target_FUFEI_discrete,
           batch_feature_tensor_target_QOE_continue,
           batch_feature_tensor_target_CHONGHE_continue,
           batch_feature_tensor_target_FUFEI_continue,
           qoe_tables, chonghe_tables, fufei_tables,
           qoe_cont_w, qoe_cont_b, chonghe_cont_w, chonghe_cont_b,
           fufei_cont_w, fufei_cont_b):
    return _impl(batch_feature_tensor_target_QOE_discrete,
                 batch_feature_tensor_target_CHONGHE_discrete,
                 batch_feature_tensor_target_FUFEI_discrete,
                 batch_feature_tensor_target_QOE_continue,
                 batch_feature_tensor_target_CHONGHE_continue,
                 batch_feature_tensor_target_FUFEI_continue,
                 qoe_tables, chonghe_tables, fufei_tables,
                 qoe_cont_w, qoe_cont_b, chonghe_cont_w, chonghe_cont_b,
                 fufei_cont_w, fufei_cont_b)


# speedup vs baseline: 2.2102x; 1.0003x over previous
import jax
import jax.numpy as jnp
from jax import lax
from jax.experimental import pallas as pl
from jax.experimental.pallas import tpu as pltpu
from jax.experimental.pallas import tpu_sc as plsc

B = 4096
N_DISC, N_CONT = 6, 4
N_FEAT = N_DISC + N_CONT
V1 = 100001
D = 32
NC, NS = 2, 16
NW = NC * NS
BW = B // NW
NK = BW // 16
NG = 3
BK = 512


def _sc_body(*refs):
    idx = refs[0]
    tab = refs[1]
    out = refs[2]
    istage = refs[3]
    ring = refs[4]
    gslab = refs[5]
    gsems = refs[6]
    wsem = refs[7]

    wid = lax.axis_index("s") * NC + lax.axis_index("c")
    base = pl.multiple_of(wid * BW, BW)

    def fire_chunk(i, k):
        cvec = istage[0, pl.ds(pl.multiple_of(k * 16, 16), 16)]
        half = lax.rem(k, 2)
        for j in range(16):
            s = cvec[j] + 1
            blk = pl.multiple_of((s // 8) * 8, 8)
            slot = pl.multiple_of(half * BW + j * 8, 8)
            pltpu.async_copy(tab.at[i, pl.ds(blk, 8)],
                             ring.at[pl.ds(slot, 8)], gsems.at[half])

    def retire_chunk(il, i, k):
        cvec = istage[0, pl.ds(pl.multiple_of(k * 16, 16), 16)]
        half = lax.rem(k, 2)
        pltpu.make_async_copy(tab.at[i, pl.ds(0, BW)],
                              ring.at[pl.ds(pl.multiple_of(half * BW, 8),
                                            BW)],
                              gsems.at[half]).wait()
        base_r = half * BW
        for j in range(16):
            s = cvec[j] + 1
            rem = lax.rem(s, 8)
            row = base_r + j * 8 + rem
            gslab[il, k * 16 + j, pl.ds(0, 16)] = ring[row, pl.ds(0, 16)]
            gslab[il, k * 16 + j, pl.ds(16, 16)] = ring[row, pl.ds(16, 16)]

    prev_dst = None
    for h in range(2):
        if prev_dst is not None:
            pltpu.make_async_copy(gslab, prev_dst, wsem).wait()

        def iloop(il, carry, h=h):
            i = h * (N_DISC // 2) + il
            pltpu.sync_copy(idx.at[i, wid], istage)
            fire_chunk(i, 0)

            def kbody(k, c):
                fire_chunk(i, k)
                retire_chunk(il, i, k - 1)
                return c

            lax.fori_loop(1, NK, kbody, None)
            retire_chunk(il, i, NK - 1)
            return carry

        lax.fori_loop(0, N_DISC // 2, iloop, None)
        dst = out.at[pl.ds(h * (N_DISC // 2), N_DISC // 2),
                     pl.ds(base, BW)]
        pltpu.async_copy(gslab, dst, wsem)
        prev_dst = dst

    pltpu.make_async_copy(gslab, prev_dst, wsem).wait()


def _tc_body(c0, c1, c2, w0, b0, w1, b1, w2, b2, o0, o1, o2):
    for c, w, bb, o in ((c0, w0, b0, o0), (c1, w1, b1, o1),
                        (c2, w2, b2, o2)):
        cv = c[:, 0, :]
        for j in range(N_CONT):
            o[:, j, :] = (cv[:, j][:, None] * w[j][None, :]
                          + bb[j][None, :])


@jax.jit
def _impl(qoe_d, ch_d, fu_d, qoe_c, ch_c, fu_c,
          qoe_tab, ch_tab, fu_tab,
          qoe_w, qoe_b, ch_w, ch_b, fu_w, fu_b):
    idxT = [d.reshape(B, N_DISC).T.reshape(N_DISC, NW, 1, BW)
            for d in (qoe_d, ch_d, fu_d)]

    mesh = plsc.VectorSubcoreMesh(core_axis_name="c", subcore_axis_name="s")
    out_t = jax.ShapeDtypeStruct((N_DISC, B, D), jnp.float32)
    scratch = (
        [pltpu.VMEM((1, BW), jnp.int32)]
        + [pltpu.VMEM((2 * BW, D), jnp.float32)]
        + [pltpu.VMEM((N_DISC // 2, BW, D), jnp.float32)]
        + [pltpu.SemaphoreType.DMA((2,)), pltpu.SemaphoreType.DMA]
    )
    run = pl.kernel(_sc_body, mesh=mesh, out_type=out_t,
                    scratch_types=scratch,
                    compiler_params=pltpu.CompilerParams(
                        use_tc_tiling_on_sc=True,
                        needs_layout_passes=False))
    grid = (B // BK,)
    cspec = pl.BlockSpec((BK, 1, N_CONT), lambda b: (b, 0, 0))
    wspec = pl.BlockSpec((N_CONT, D), lambda b: (0, 0))
    ospec = pl.BlockSpec((BK, N_CONT, D), lambda b: (b, 0, 0))
    conts = pl.pallas_call(
        _tc_body,
        grid=grid,
        in_specs=[cspec] * 3 + [wspec] * 6,
        out_specs=[ospec] * 3,
        out_shape=[jax.ShapeDtypeStruct((B, N_CONT, D), jnp.float32)] * 3,
    )(qoe_c, ch_c, fu_c, qoe_w, qoe_b, ch_w, ch_b, fu_w, fu_b)

    discs = [run(idxT[g], t)
             for g, t in enumerate((qoe_tab, ch_tab, fu_tab))]
    return tuple(
        jnp.concatenate([d.transpose(1, 0, 2), c], axis=1)
        .reshape(B, 1, N_FEAT, D)
        for d, c in zip(discs, conts))


def kernel(batch_feature_tensor_target_QOE_discrete,
           batch_feature_tensor_target_CHONGHE_discrete,
           batch_feature_tensor_target_FUFEI_discrete,
           batch_feature_tensor_target_QOE_continue,
           batch_feature_tensor_target_CHONGHE_continue,
           batch_feature_tensor_target_FUFEI_continue,
           qoe_tables, chonghe_tables, fufei_tables,
           qoe_cont_w, qoe_cont_b, chonghe_cont_w, chonghe_cont_b,
           fufei_cont_w, fufei_cont_b):
    return _impl(batch_feature_tensor_target_QOE_discrete,
                 batch_feature_tensor_target_CHONGHE_discrete,
                 batch_feature_tensor_target_FUFEI_discrete,
                 batch_feature_tensor_target_QOE_continue,
                 batch_feature_tensor_target_CHONGHE_continue,
                 batch_feature_tensor_target_FUFEI_continue,
                 qoe_tables, chonghe_tables, fufei_tables,
                 qoe_cont_w, qoe_cont_b, chonghe_cont_w, chonghe_cont_b,
                 fufei_cont_w, fufei_cont_b)
